# baseline (device time: 32120 ns/iter reference)
import jax
import jax.numpy as jnp
from jax import lax
from jax.experimental import pallas as pl
from jax.experimental.pallas import tpu as pltpu

N_DEV = 4


def kernel(x, W1, W2):
    m, k = x.shape
    n = W2.shape[1]

    def body(x_ref, w1_ref, w2_ref, out_ref, comm_ref, send_sems, recv_sems):
        my = lax.axis_index("i")
        left = lax.rem(my + N_DEV - 1, N_DEV)
        right = lax.rem(my + 1, N_DEV)

        xb = x_ref[...].astype(jnp.bfloat16)
        w1b = w1_ref[...].astype(jnp.bfloat16)
        h = jnp.dot(xb, w1b, preferred_element_type=jnp.float32)
        hb = jnp.maximum(h, 0.0).astype(jnp.bfloat16)
        w2b = w2_ref[...].astype(jnp.bfloat16)
        partial = jnp.dot(hb, w2b, preferred_element_type=jnp.float32)

        out_ref[...] = partial
        comm_ref[N_DEV - 1] = partial.astype(jnp.bfloat16)

        barrier_sem = pltpu.get_barrier_semaphore()
        for nbr in [left, right]:
            pl.semaphore_signal(
                barrier_sem, inc=1,
                device_id=(nbr,), device_id_type=pl.DeviceIdType.MESH,
            )
        pl.semaphore_wait(barrier_sem, 2)

        for hop in range(N_DEV - 1):
            src_slot = N_DEV - 1 if hop == 0 else hop - 1
            rdma = pltpu.make_async_remote_copy(
                src_ref=comm_ref.at[src_slot],
                dst_ref=comm_ref.at[hop],
                send_sem=send_sems.at[hop],
                recv_sem=recv_sems.at[hop],
                device_id=(right,),
                device_id_type=pl.DeviceIdType.MESH,
            )
            rdma.start()
            rdma.wait()
            out_ref[...] += comm_ref[hop].astype(jnp.float32)

    return pl.pallas_call(
        body,
        out_shape=jax.ShapeDtypeStruct((m, n), jnp.float32),
        in_specs=[
            pl.BlockSpec(memory_space=pltpu.VMEM),
            pl.BlockSpec(memory_space=pltpu.VMEM),
            pl.BlockSpec(memory_space=pltpu.VMEM),
        ],
        out_specs=pl.BlockSpec(memory_space=pltpu.VMEM),
        scratch_shapes=[
            pltpu.VMEM((N_DEV, m, n), jnp.bfloat16),
            pltpu.SemaphoreType.DMA((N_DEV - 1,)),
            pltpu.SemaphoreType.DMA((N_DEV - 1,)),
        ],
        compiler_params=pltpu.CompilerParams(collective_id=0),
    )(x, W1, W2)


# device time: 18463 ns/iter; 1.7397x vs baseline; 1.7397x over previous
import jax
import jax.numpy as jnp
from jax import lax
from jax.experimental import pallas as pl
from jax.experimental.pallas import tpu as pltpu

N_DEV = 4


def kernel(x, W1, W2):
    m, k = x.shape
    n = W2.shape[1]
    mq = m // N_DEV

    def body(x_ref, w1_ref, w2_ref, out_ref,
             part_buf, rs_buf, ag_src, ag_buf,
             rs_send, rs_recv, ag_send, ag_recv):
        my = lax.axis_index("i")

        barrier_sem = pltpu.get_barrier_semaphore()
        for t in range(N_DEV - 1):
            pl.semaphore_signal(
                barrier_sem, inc=1,
                device_id=(lax.rem(my + 1 + t, N_DEV),),
                device_id_type=pl.DeviceIdType.MESH,
            )

        xb = x_ref[...].astype(jnp.bfloat16)
        w1b = w1_ref[...].astype(jnp.bfloat16)
        h = jnp.dot(xb, w1b, preferred_element_type=jnp.float32)
        hb = jnp.maximum(h, 0.0).astype(jnp.bfloat16)
        w2b = w2_ref[...].astype(jnp.bfloat16)
        partial = jnp.dot(hb, w2b, preferred_element_type=jnp.float32)

        part_buf[...] = partial.astype(jnp.bfloat16).reshape(N_DEV, mq, n)

        pl.semaphore_wait(barrier_sem, N_DEV - 1)

        rs = []
        for t in range(N_DEV - 1):
            dst = lax.rem(my + 1 + t, N_DEV)
            rdma = pltpu.make_async_remote_copy(
                src_ref=part_buf.at[dst],
                dst_ref=rs_buf.at[2 - t],
                send_sem=rs_send.at[t],
                recv_sem=rs_recv.at[2 - t],
                device_id=(dst,),
                device_id_type=pl.DeviceIdType.MESH,
            )
            rdma.start()
            rs.append(rdma)
        for rdma in rs:
            rdma.wait()

        red = part_buf[my].astype(jnp.float32)
        for s in range(N_DEV - 1):
            red = red + rs_buf[s].astype(jnp.float32)
        out_ref[pl.ds(my * mq, mq), :] = red
        ag_src[...] = red.astype(jnp.bfloat16)

        ag = []
        for t in range(N_DEV - 1):
            dst = lax.rem(my + 1 + t, N_DEV)
            rdma = pltpu.make_async_remote_copy(
                src_ref=ag_src,
                dst_ref=ag_buf.at[2 - t],
                send_sem=ag_send.at[t],
                recv_sem=ag_recv.at[2 - t],
                device_id=(dst,),
                device_id_type=pl.DeviceIdType.MESH,
            )
            rdma.start()
            ag.append(rdma)
        for rdma in ag:
            rdma.wait()

        for s in range(N_DEV - 1):
            src_dev = lax.rem(my + 1 + s, N_DEV)
            out_ref[pl.ds(src_dev * mq, mq), :] = ag_buf[s].astype(jnp.float32)

    return pl.pallas_call(
        body,
        out_shape=jax.ShapeDtypeStruct((m, n), jnp.float32),
        in_specs=[
            pl.BlockSpec(memory_space=pltpu.VMEM),
            pl.BlockSpec(memory_space=pltpu.VMEM),
            pl.BlockSpec(memory_space=pltpu.VMEM),
        ],
        out_specs=pl.BlockSpec(memory_space=pltpu.VMEM),
        scratch_shapes=[
            pltpu.VMEM((N_DEV, mq, n), jnp.bfloat16),
            pltpu.VMEM((N_DEV - 1, mq, n), jnp.bfloat16),
            pltpu.VMEM((mq, n), jnp.bfloat16),
            pltpu.VMEM((N_DEV - 1, mq, n), jnp.bfloat16),
            pltpu.SemaphoreType.DMA((N_DEV - 1,)),
            pltpu.SemaphoreType.DMA((N_DEV - 1,)),
            pltpu.SemaphoreType.DMA((N_DEV - 1,)),
            pltpu.SemaphoreType.DMA((N_DEV - 1,)),
        ],
        compiler_params=pltpu.CompilerParams(collective_id=0),
    )(x, W1, W2)


# device time: 17927 ns/iter; 1.7917x vs baseline; 1.0299x over previous
import jax
import jax.numpy as jnp
from jax import lax
from jax.experimental import pallas as pl
from jax.experimental.pallas import tpu as pltpu

N_DEV = 4


def kernel(x, W1, W2):
    m, k = x.shape
    n = W2.shape[1]
    mq = m // N_DEV

    def body(x_ref, w1_ref, w2_ref, out_ref,
             part_buf, rs_buf, ag_src, ag_buf,
             rs_send, rs_recv, ag_send, ag_recv):
        my = lax.axis_index("i")

        barrier_sem = pltpu.get_barrier_semaphore()
        for t in range(N_DEV - 1):
            pl.semaphore_signal(
                barrier_sem, inc=1,
                device_id=(lax.rem(my + 1 + t, N_DEV),),
                device_id_type=pl.DeviceIdType.MESH,
            )

        w1b = w1_ref[...].astype(jnp.bfloat16)
        w2b = w2_ref[...].astype(jnp.bfloat16)

        def quarter(dst):
            xc = x_ref[pl.ds(dst * mq, mq), :].astype(jnp.bfloat16)
            hc = jnp.dot(xc, w1b, preferred_element_type=jnp.float32)
            hc = jnp.maximum(hc, 0.0).astype(jnp.bfloat16)
            return jnp.dot(hc, w2b, preferred_element_type=jnp.float32)

        pl.semaphore_wait(barrier_sem, N_DEV - 1)

        rs = []
        for t in range(N_DEV - 1):
            dst = lax.rem(my + 1 + t, N_DEV)
            part_buf[t] = quarter(dst).astype(jnp.bfloat16)
            rdma = pltpu.make_async_remote_copy(
                src_ref=part_buf.at[t],
                dst_ref=rs_buf.at[2 - t],
                send_sem=rs_send.at[t],
                recv_sem=rs_recv.at[2 - t],
                device_id=(dst,),
                device_id_type=pl.DeviceIdType.MESH,
            )
            rdma.start()
            rs.append(rdma)

        red = quarter(my)
        for rdma in rs:
            rdma.wait()
        for s in range(N_DEV - 1):
            red = red + rs_buf[s].astype(jnp.float32)
        out_ref[pl.ds(my * mq, mq), :] = red
        ag_src[...] = red.astype(jnp.bfloat16)

        ag = []
        for t in range(N_DEV - 1):
            dst = lax.rem(my + 1 + t, N_DEV)
            rdma = pltpu.make_async_remote_copy(
                src_ref=ag_src,
                dst_ref=ag_buf.at[2 - t],
                send_sem=ag_send.at[t],
                recv_sem=ag_recv.at[2 - t],
                device_id=(dst,),
                device_id_type=pl.DeviceIdType.MESH,
            )
            rdma.start()
            ag.append(rdma)
        for rdma in ag:
            rdma.wait()

        for s in range(N_DEV - 1):
            src_dev = lax.rem(my + 1 + s, N_DEV)
            out_ref[pl.ds(src_dev * mq, mq), :] = ag_buf[s].astype(jnp.float32)

    return pl.pallas_call(
        body,
        out_shape=jax.ShapeDtypeStruct((m, n), jnp.float32),
        in_specs=[
            pl.BlockSpec(memory_space=pltpu.VMEM),
            pl.BlockSpec(memory_space=pltpu.VMEM),
            pl.BlockSpec(memory_space=pltpu.VMEM),
        ],
        out_specs=pl.BlockSpec(memory_space=pltpu.VMEM),
        scratch_shapes=[
            pltpu.VMEM((N_DEV - 1, mq, n), jnp.bfloat16),
            pltpu.VMEM((N_DEV - 1, mq, n), jnp.bfloat16),
            pltpu.VMEM((mq, n), jnp.bfloat16),
            pltpu.VMEM((N_DEV - 1, mq, n), jnp.bfloat16),
            pltpu.SemaphoreType.DMA((N_DEV - 1,)),
            pltpu.SemaphoreType.DMA((N_DEV - 1,)),
            pltpu.SemaphoreType.DMA((N_DEV - 1,)),
            pltpu.SemaphoreType.DMA((N_DEV - 1,)),
        ],
        compiler_params=pltpu.CompilerParams(collective_id=0),
    )(x, W1, W2)
